# baseline (speedup 1.0000x reference)
import functools

import jax
import jax.numpy as jnp
from jax import lax
from jax.experimental import pallas as pl
from jax.experimental.pallas import tpu as pltpu
from jax.experimental.pallas import tpu_sc as plsc

_LANES = 16


def _build_sc_pe_add(s_sc, batch, d_model, n_workers, s_chunk):
    mesh = plsc.VectorSubcoreMesh(core_axis_name="c", subcore_axis_name="s")
    info = plsc.get_sparse_core_info()
    nc = info.num_cores
    pos_per_worker = s_sc // n_workers
    n_chunks = pos_per_worker // s_chunk
    vecs = d_model // _LANES

    @functools.partial(
        pl.kernel,
        mesh=mesh,
        out_type=jax.ShapeDtypeStruct((s_sc * batch, d_model), jnp.float32),
        scratch_types=[
            pltpu.VMEM((s_chunk * batch, d_model), jnp.float32),
            pltpu.VMEM((s_chunk, d_model), jnp.float32),
        ],
    )
    def pe_add(x_hbm, pe_hbm, out_hbm, xbuf, pebuf):
        wid = lax.axis_index("s") * nc + lax.axis_index("c")
        pos0 = wid * pos_per_worker

        def chunk_body(c, carry):
            p0 = pos0 + c * s_chunk
            r0 = p0 * batch
            pltpu.sync_copy(x_hbm.at[pl.ds(r0, s_chunk * batch)], xbuf)
            pltpu.sync_copy(pe_hbm.at[pl.ds(p0, s_chunk)], pebuf)

            def pos_body(p, carry2):
                for k in range(vecs):
                    v = pebuf[p, pl.ds(k * _LANES, _LANES)]
                    for b in range(batch):
                        plsc.addupdate(
                            xbuf.at[p * batch + b, pl.ds(k * _LANES, _LANES)], v
                        )
                return carry2

            lax.fori_loop(0, s_chunk, pos_body, 0)
            pltpu.sync_copy(xbuf, out_hbm.at[pl.ds(r0, s_chunk * batch)])
            return carry

        lax.fori_loop(0, n_chunks, chunk_body, 0)

    return pe_add


def _tc_body(x_ref, pe_ref, o_ref):
    o_ref[...] = x_ref[...] + pe_ref[...][:, None, :]


def kernel(x, pe_weight):
    seq, batch, d_model = x.shape
    s_sc = 1024
    p = 256
    off = s_sc // p

    sc_pe_add = _build_sc_pe_add(s_sc, batch, d_model, n_workers=32, s_chunk=16)
    sc_rows = sc_pe_add(
        x.reshape(seq * batch, d_model)[: s_sc * batch], pe_weight[:s_sc]
    )
    sc_out = sc_rows.reshape(s_sc, batch, d_model)

    tc_out = pl.pallas_call(
        _tc_body,
        grid=((seq - s_sc) // p,),
        in_specs=[
            pl.BlockSpec((p, batch, d_model), lambda i: (i + off, 0, 0)),
            pl.BlockSpec((p, d_model), lambda i: (i + off, 0)),
        ],
        out_specs=pl.BlockSpec((p, batch, d_model), lambda i: (i + off, 0, 0)),
        out_shape=jax.ShapeDtypeStruct((seq, batch, d_model), jnp.float32),
    )(x, pe_weight)

    return lax.dynamic_update_slice(tc_out, sc_out, (0, 0, 0))


# speedup vs baseline: 1.9226x; 1.9226x over previous
import jax
import jax.numpy as jnp
from jax.experimental import pallas as pl


def _tc_body(x_ref, pe_ref, o_ref):
    o_ref[...] = x_ref[...] + pe_ref[...][:, None, :]


def kernel(x, pe_weight):
    seq, batch, d_model = x.shape
    p = 512
    out = pl.pallas_call(
        _tc_body,
        grid=(seq // p,),
        in_specs=[
            pl.BlockSpec((p, batch, d_model), lambda i: (i, 0, 0)),
            pl.BlockSpec((p, d_model), lambda i: (i, 0)),
        ],
        out_specs=pl.BlockSpec((p, batch, d_model), lambda i: (i, 0, 0)),
        out_shape=jax.ShapeDtypeStruct((seq, batch, d_model), jnp.float32),
    )(x, pe_weight)
    return out
